# baseline (speedup 1.0000x reference)
import functools

import jax
import jax.numpy as jnp
from jax import lax
from jax.experimental import pallas as pl
from jax.experimental.pallas import tpu as pltpu
from jax.experimental.pallas import tpu_sc as plsc

_B = 16384
_V = 100000
_D = 64
_L = 16
_NC = 2
_NS = 16
_NW = _NC * _NS
_BPW = _B // _NW
_CHUNK = 128
_NCHUNK = _BPW // _CHUNK
_NBLK = _D // _L
_VHI = 51200
_CBHI = 2048
_VHU = 53248
_CBHU = 4096

_SC_PARAMS = pltpu.CompilerParams(use_tc_tiling_on_sc=True,
                                  needs_layout_passes=False)
_MESH = plsc.VectorSubcoreMesh(core_axis_name="c", subcore_axis_name="s")


def _pack_specs(vh, cbh):
    nblks = vh // cbh
    last_in_blk = pl.cdiv(_V, cbh) - 1
    lo = pl.BlockSpec((_D, cbh), lambda j: (0, j))
    hi = pl.BlockSpec(
        (_D, cbh), lambda j: (0, jnp.minimum(j + nblks, last_in_blk)))
    out = pl.BlockSpec((cbh, 2 * _D), lambda j: (j, 0))
    shape = jax.ShapeDtypeStruct((vh, 2 * _D), jnp.float32)
    return nblks, lo, hi, out, shape


def _tc_pack_item(item):
    def body(itl, ith, i_ref):
        i_ref[...] = jnp.concatenate([itl[...].T, ith[...].T], axis=1)

    nblks, lo, hi, out, shape = _pack_specs(_VHI, _CBHI)
    return pl.pallas_call(
        body, grid=(nblks,),
        in_specs=[lo, hi],
        out_specs=out, out_shape=shape,
    )(item.T, item.T)


def _tc_pack_user(t0, t1, t2):
    def body(a0l, a0h, a1l, a1h, a2l, a2h, u_ref):
        ul = a0l[...] + 0.5 * (a1l[...] + a2l[...])
        uh = a0h[...] + 0.5 * (a1h[...] + a2h[...])
        u_ref[...] = jnp.concatenate([ul.T, uh.T], axis=1)

    nblks, lo, hi, out, shape = _pack_specs(_VHU, _CBHU)
    return pl.pallas_call(
        body, grid=(nblks,),
        in_specs=[lo, hi] * 3,
        out_specs=out, out_shape=shape,
    )(t0.T, t0.T, t1.T, t1.T, t2.T, t2.T)


_ROWBUF = pltpu.VMEM((_CHUNK, 2 * _D), jnp.float32)


def _sc_item(pos, neg, itab):
    @functools.partial(
        pl.kernel,
        out_type=(
            jax.ShapeDtypeStruct((_B // 2, 2 * _D), jnp.float32),
            jax.ShapeDtypeStruct((_NW, _L), jnp.float32),
        ),
        mesh=_MESH,
        compiler_params=_SC_PARAMS,
        scratch_types=[
            pltpu.VMEM((_BPW,), jnp.int32),
            pltpu.VMEM((_BPW,), jnp.int32),
            pltpu.VMEM((_BPW,), jnp.int32),
            pltpu.VMEM((_BPW,), jnp.int32),
            _ROWBUF, _ROWBUF,
            _ROWBUF, _ROWBUF,
            pltpu.VMEM((_BPW // 2, 2 * _D), jnp.float32),
            pltpu.VMEM((_L,), jnp.float32),
            pltpu.SemaphoreType.DMA,
            pltpu.SemaphoreType.DMA,
        ],
    )
    def body(pos_h, neg_h, itab_h, dv_h, r_h,
             pidx, nidx, mpidx, mnidx,
             pva, nva, pvb, nvb, dv, regv, sema, semb):
        wid = lax.axis_index("s") * _NC + lax.axis_index("c")
        base = wid * _BPW
        pltpu.sync_copy(pos_h.at[pl.ds(base, _BPW)], pidx)
        pltpu.sync_copy(neg_h.at[pl.ds(base, _BPW)], nidx)

        lane = lax.iota(jnp.int32, _L)

        def m_body(g, carry):
            sl16 = pl.ds(g * _L, _L)
            for src, dst in ((pidx, mpidx), (nidx, mnidx)):
                v = src[sl16]
                dst[sl16] = v - jnp.where(v >= _VHI, _VHI, 0)
            return carry

        lax.fori_loop(0, _BPW // _L, m_body, 0)

        def fire(ci, pv, nv, sem):
            isl = pl.ds(ci * _CHUNK, _CHUNK)
            pltpu.async_copy(itab_h.at[mpidx.at[isl]], pv, sem)
            pltpu.async_copy(itab_h.at[mnidx.at[isl]], nv, sem)

        def drain(pv, nv, sem):
            pltpu.make_async_copy(itab_h.at[pl.ds(0, _CHUNK)], pv, sem).wait()
            pltpu.make_async_copy(itab_h.at[pl.ds(0, _CHUNK)], nv, sem).wait()

        def compute(ci, pv, nv, regacc):
            off = ci * _CHUNK

            def row_body(r, reg):
                rsplat = jnp.full((_L,), r, jnp.int32)
                isplat = rsplat + off
                cp = jnp.where(plsc.load_gather(pidx, [isplat]) >= _VHI,
                               _D, 0) + lane
                cn = jnp.where(plsc.load_gather(nidx, [isplat]) >= _VHI,
                               _D, 0) + lane
                drow = isplat >> 1
                dcol = (isplat & 1) * _D + lane
                for k in range(_NBLK):
                    pp = plsc.load_gather(pv, [rsplat, cp + k * _L])
                    nn = plsc.load_gather(nv, [rsplat, cn + k * _L])
                    plsc.store_scatter(dv, [drow, dcol + k * _L], nn - pp)
                    reg = reg + (pp * pp + nn * nn)
                return reg

            return plsc.parallel_loop(0, _CHUNK, unroll=4,
                                      carry=regacc)(row_body)

        regacc = jnp.zeros((_L,), jnp.float32)
        fire(0, pva, nva, sema)
        fire(1, pvb, nvb, semb)
        drain(pva, nva, sema)
        regacc = compute(0, pva, nva, regacc)
        fire(2, pva, nva, sema)
        drain(pvb, nvb, semb)
        regacc = compute(1, pvb, nvb, regacc)
        fire(3, pvb, nvb, semb)
        drain(pva, nva, sema)
        regacc = compute(2, pva, nva, regacc)
        drain(pvb, nvb, semb)
        regacc = compute(3, pvb, nvb, regacc)

        pltpu.sync_copy(dv, dv_h.at[pl.ds(wid * (_BPW // 2), _BPW // 2)])
        regv[...] = regacc
        pltpu.sync_copy(regv, r_h.at[wid])

    return body(pos, neg, itab)


def _sc_user(users, utab, dv2):
    dchunk = pltpu.VMEM((_CHUNK // 2, 2 * _D), jnp.float32)

    @functools.partial(
        pl.kernel,
        out_type=(
            jax.ShapeDtypeStruct((_L, _B), jnp.float32),
            jax.ShapeDtypeStruct((_NW, _L), jnp.float32),
        ),
        mesh=_MESH,
        compiler_params=_SC_PARAMS,
        scratch_types=[
            pltpu.VMEM((_BPW,), jnp.int32),
            pltpu.VMEM((_BPW,), jnp.int32),
            _ROWBUF, dchunk,
            _ROWBUF, dchunk,
            pltpu.VMEM((_L, _BPW), jnp.float32),
            pltpu.VMEM((_L,), jnp.float32),
            pltpu.SemaphoreType.DMA,
            pltpu.SemaphoreType.DMA,
        ],
    )
    def body(users_h, utab_h, dv2_h, q_h, r_h,
             uidx, muidx, uva, dva, uvb, dvb, qv, regv, sema, semb):
        wid = lax.axis_index("s") * _NC + lax.axis_index("c")
        base = wid * _BPW
        pltpu.sync_copy(users_h.at[pl.ds(base, _BPW)], uidx)

        lane = lax.iota(jnp.int32, _L)

        def m_body(g, carry):
            sl16 = pl.ds(g * _L, _L)
            v = uidx[sl16]
            muidx[sl16] = v - jnp.where(v >= _VHU, _VHU, 0)
            return carry

        lax.fori_loop(0, _BPW // _L, m_body, 0)

        def fire(ci, uv, dvv, sem):
            pltpu.async_copy(
                utab_h.at[muidx.at[pl.ds(ci * _CHUNK, _CHUNK)]], uv, sem)
            pltpu.async_copy(
                dv2_h.at[pl.ds(wid * (_BPW // 2) + ci * (_CHUNK // 2),
                               _CHUNK // 2)],
                dvv, sem)

        def drain(uv, dvv, sem):
            pltpu.make_async_copy(utab_h.at[pl.ds(0, _CHUNK)], uv, sem).wait()
            pltpu.make_async_copy(
                dv2_h.at[pl.ds(0, _CHUNK // 2)], dvv, sem).wait()

        def compute(ci, uv, dvv, regacc):
            off = ci * _CHUNK

            def row_body(r, reg):
                rsplat = jnp.full((_L,), r, jnp.int32)
                isplat = rsplat + off
                cu = jnp.where(plsc.load_gather(uidx, [isplat]) >= _VHU,
                               _D, 0) + lane
                drow = rsplat >> 1
                dcol = (rsplat & 1) * _D + lane
                q = None
                for k in range(_NBLK):
                    u = plsc.load_gather(uv, [rsplat, cu + k * _L])
                    dd = plsc.load_gather(dvv, [drow, dcol + k * _L])
                    term = u * dd
                    q = term if q is None else q + term
                    reg = reg + u * u
                plsc.store_scatter(qv, [lane, isplat], q)
                return reg

            return plsc.parallel_loop(0, _CHUNK, unroll=4,
                                      carry=regacc)(row_body)

        regacc = jnp.zeros((_L,), jnp.float32)
        fire(0, uva, dva, sema)
        fire(1, uvb, dvb, semb)
        drain(uva, dva, sema)
        regacc = compute(0, uva, dva, regacc)
        fire(2, uva, dva, sema)
        drain(uvb, dvb, semb)
        regacc = compute(1, uvb, dvb, regacc)
        fire(3, uvb, dvb, semb)
        drain(uva, dva, sema)
        regacc = compute(2, uva, dva, regacc)
        drain(uvb, dvb, semb)
        regacc = compute(3, uvb, dvb, regacc)

        pltpu.sync_copy(qv, q_h.at[:, pl.ds(base, _BPW)])
        regv[...] = regacc
        pltpu.sync_copy(regv, r_h.at[wid])

    return body(users, utab, dv2)


def _tc_finalize(q, r1, r2):
    def tc_body(q_ref, r1_ref, r2_ref, loss_ref, reg_ref):
        d = jnp.sum(q_ref[...], axis=0)
        loss_ref[0, 0] = jnp.mean(jax.nn.softplus(d))
        reg_ref[0, 0] = (0.5 * (jnp.sum(r1_ref[...]) + jnp.sum(r2_ref[...]))
                         / float(_B))

    loss, reg = pl.pallas_call(
        tc_body,
        out_shape=(
            jax.ShapeDtypeStruct((1, 1), jnp.float32),
            jax.ShapeDtypeStruct((1, 1), jnp.float32),
        ),
        out_specs=(
            pl.BlockSpec(memory_space=pltpu.SMEM),
            pl.BlockSpec(memory_space=pltpu.SMEM),
        ),
    )(q, r1, r2)
    return loss[0, 0], reg[0, 0]


def kernel(users, pos, neg, user_table_0, user_table_1, user_table_2,
           item_table):
    itab = _tc_pack_item(item_table)
    dv2, r1 = _sc_item(pos, neg, itab)
    utab = _tc_pack_user(user_table_0, user_table_1, user_table_2)
    q, r2 = _sc_user(users, utab, dv2)
    loss, reg_loss = _tc_finalize(q, r1, r2)
    return (loss, reg_loss)


# speedup vs baseline: 1.0818x; 1.0818x over previous
import functools

import jax
import jax.numpy as jnp
from jax import lax
from jax.experimental import pallas as pl
from jax.experimental.pallas import tpu as pltpu
from jax.experimental.pallas import tpu_sc as plsc

_B = 16384
_V = 100000
_D = 64
_L = 16
_NC = 2
_NS = 16
_NW = _NC * _NS
_BPW = _B // _NW
_CHUNK = 128
_NCHUNK = _BPW // _CHUNK
_NBLK = _D // _L
_CBH = 4096
_VH = 53248


def _tc_pack(t0, t1, t2, item):
    t0t, t1t, t2t, itt = t0.T, t1.T, t2.T, item.T

    def pack_body(a0l, a0h, a1l, a1h, a2l, a2h, itl, ith, u_ref, i_ref):
        ul = a0l[...] + 0.5 * (a1l[...] + a2l[...])
        uh = a0h[...] + 0.5 * (a1h[...] + a2h[...])
        u_ref[...] = jnp.concatenate([ul.T, uh.T], axis=1)
        i_ref[...] = jnp.concatenate([itl[...].T, ith[...].T], axis=1)

    nblk = _VH // _CBH
    last_in_blk = pl.cdiv(_V, _CBH) - 1
    lo_spec = pl.BlockSpec((_D, _CBH), lambda j: (0, j))
    hi_spec = pl.BlockSpec(
        (_D, _CBH), lambda j: (0, jnp.minimum(j + nblk, last_in_blk)))
    out_spec = pl.BlockSpec((_CBH, 2 * _D), lambda j: (j, 0))
    return pl.pallas_call(
        pack_body,
        grid=(nblk,),
        in_specs=[lo_spec, hi_spec] * 4,
        out_specs=[out_spec] * 2,
        out_shape=[jax.ShapeDtypeStruct((_VH, 2 * _D), jnp.float32)] * 2,
    )(t0t, t0t, t1t, t1t, t2t, t2t, itt, itt)


def _sc_gather_score(users, pos, neg, utab, itab):
    mesh = plsc.VectorSubcoreMesh(core_axis_name="c", subcore_axis_name="s")

    rowbuf = pltpu.VMEM((_CHUNK, 2 * _D), jnp.float32)

    @functools.partial(
        pl.kernel,
        out_type=(
            jax.ShapeDtypeStruct((_L, _B), jnp.float32),
            jax.ShapeDtypeStruct((_NW, _L), jnp.float32),
        ),
        mesh=mesh,
        compiler_params=pltpu.CompilerParams(use_tc_tiling_on_sc=True,
                                             needs_layout_passes=False),
        scratch_types=[
            pltpu.VMEM((_BPW,), jnp.int32),
            pltpu.VMEM((_BPW,), jnp.int32),
            pltpu.VMEM((_BPW,), jnp.int32),
            pltpu.VMEM((_BPW,), jnp.int32),
            pltpu.VMEM((_BPW,), jnp.int32),
            pltpu.VMEM((_BPW,), jnp.int32),
            rowbuf, rowbuf, rowbuf,
            rowbuf, rowbuf, rowbuf,
            pltpu.VMEM((_L, _BPW), jnp.float32),
            pltpu.VMEM((_L,), jnp.float32),
            pltpu.SemaphoreType.DMA,
            pltpu.SemaphoreType.DMA,
        ],
    )
    def sc_body(users_h, pos_h, neg_h, utab_h, itab_h,
                q_h, r_h,
                uidx, pidx, nidx, muidx, mpidx, mnidx,
                uva, pva, nva, uvb, pvb, nvb,
                qv, regv, sema, semb):
        wid = lax.axis_index("s") * _NC + lax.axis_index("c")
        base = wid * _BPW
        pltpu.sync_copy(users_h.at[pl.ds(base, _BPW)], uidx)
        pltpu.sync_copy(pos_h.at[pl.ds(base, _BPW)], pidx)
        pltpu.sync_copy(neg_h.at[pl.ds(base, _BPW)], nidx)

        lane = lax.iota(jnp.int32, _L)

        def m_body(g, carry):
            sl16 = pl.ds(g * _L, _L)
            for src, dst in ((uidx, muidx), (pidx, mpidx), (nidx, mnidx)):
                v = src[sl16]
                dst[sl16] = v - jnp.where(v >= _VH, _VH, 0)
            return carry

        lax.fori_loop(0, _BPW // _L, m_body, 0)

        def fire(ci, uv, pv, nv, sem):
            isl = pl.ds(ci * _CHUNK, _CHUNK)
            pltpu.async_copy(utab_h.at[muidx.at[isl]], uv, sem)
            pltpu.async_copy(itab_h.at[mpidx.at[isl]], pv, sem)
            pltpu.async_copy(itab_h.at[mnidx.at[isl]], nv, sem)

        def drain(uv, pv, nv, sem):
            pltpu.make_async_copy(utab_h.at[pl.ds(0, _CHUNK)], uv, sem).wait()
            pltpu.make_async_copy(itab_h.at[pl.ds(0, _CHUNK)], pv, sem).wait()
            pltpu.make_async_copy(itab_h.at[pl.ds(0, _CHUNK)], nv, sem).wait()

        def compute(ci, uv, pv, nv, regacc):
            off = ci * _CHUNK

            def row_body(r, reg):
                rsplat = jnp.full((_L,), r, jnp.int32)
                isplat = rsplat + off
                cu = jnp.where(plsc.load_gather(uidx, [isplat]) >= _VH,
                               _D, 0) + lane
                cp = jnp.where(plsc.load_gather(pidx, [isplat]) >= _VH,
                               _D, 0) + lane
                cn = jnp.where(plsc.load_gather(nidx, [isplat]) >= _VH,
                               _D, 0) + lane
                q = None
                for k in range(_NBLK):
                    u = plsc.load_gather(uv, [rsplat, cu + k * _L])
                    pp = plsc.load_gather(pv, [rsplat, cp + k * _L])
                    nn = plsc.load_gather(nv, [rsplat, cn + k * _L])
                    term = u * (nn - pp)
                    q = term if q is None else q + term
                    reg = reg + (u * u + pp * pp + nn * nn)
                plsc.store_scatter(qv, [lane, isplat], q)
                return reg

            return plsc.parallel_loop(0, _CHUNK, unroll=8,
                                      carry=regacc)(row_body)

        regacc = jnp.zeros((_L,), jnp.float32)
        fire(0, uva, pva, nva, sema)
        fire(1, uvb, pvb, nvb, semb)
        drain(uva, pva, nva, sema)
        regacc = compute(0, uva, pva, nva, regacc)
        fire(2, uva, pva, nva, sema)
        drain(uvb, pvb, nvb, semb)
        regacc = compute(1, uvb, pvb, nvb, regacc)
        fire(3, uvb, pvb, nvb, semb)
        drain(uva, pva, nva, sema)
        regacc = compute(2, uva, pva, nva, regacc)
        drain(uvb, pvb, nvb, semb)
        regacc = compute(3, uvb, pvb, nvb, regacc)

        pltpu.sync_copy(qv, q_h.at[:, pl.ds(base, _BPW)])
        regv[...] = regacc
        pltpu.sync_copy(regv, r_h.at[wid])

    return sc_body(users, pos, neg, utab, itab)


def _tc_finalize(q, r):
    def tc_body(q_ref, r_ref, loss_ref, reg_ref):
        d = jnp.sum(q_ref[...], axis=0)
        loss_ref[0, 0] = jnp.mean(jax.nn.softplus(d))
        reg_ref[0, 0] = 0.5 * jnp.sum(r_ref[...]) / float(_B)

    loss, reg = pl.pallas_call(
        tc_body,
        out_shape=(
            jax.ShapeDtypeStruct((1, 1), jnp.float32),
            jax.ShapeDtypeStruct((1, 1), jnp.float32),
        ),
        out_specs=(
            pl.BlockSpec(memory_space=pltpu.SMEM),
            pl.BlockSpec(memory_space=pltpu.SMEM),
        ),
    )(q, r)
    return loss[0, 0], reg[0, 0]


def kernel(users, pos, neg, user_table_0, user_table_1, user_table_2,
           item_table):
    utab, itab = _tc_pack(user_table_0, user_table_1, user_table_2,
                          item_table)
    q, r = _sc_gather_score(users, pos, neg, utab, itab)
    loss, reg_loss = _tc_finalize(q, r)
    return (loss, reg_loss)


# speedup vs baseline: 1.1722x; 1.0836x over previous
import functools

import jax
import jax.numpy as jnp
from jax import lax
from jax.experimental import pallas as pl
from jax.experimental.pallas import tpu as pltpu
from jax.experimental.pallas import tpu_sc as plsc

_B = 16384
_V = 100000
_D = 64
_L = 16
_NC = 2
_NS = 16
_NW = _NC * _NS
_BPW = _B // _NW
_CHUNK = 128
_NCHUNK = _BPW // _CHUNK
_NBLK = _D // _L
_CBH = 4096
_VH = 53248


def _tc_pack(t0, t1, t2, item):
    t0t, t1t, t2t, itt = t0.T, t1.T, t2.T, item.T

    def pack_body(a0l, a0h, a1l, a1h, a2l, a2h, itl, ith, u_ref, i_ref):
        ul = a0l[...] + 0.5 * (a1l[...] + a2l[...])
        uh = a0h[...] + 0.5 * (a1h[...] + a2h[...])
        u_ref[...] = jnp.concatenate([ul.T, uh.T], axis=1)
        i_ref[...] = jnp.concatenate([itl[...].T, ith[...].T], axis=1)

    nblk = _VH // _CBH
    last_in_blk = pl.cdiv(_V, _CBH) - 1
    lo_spec = pl.BlockSpec((_D, _CBH), lambda j: (0, j))
    hi_spec = pl.BlockSpec(
        (_D, _CBH), lambda j: (0, jnp.minimum(j + nblk, last_in_blk)))
    out_spec = pl.BlockSpec((_CBH, 2 * _D), lambda j: (j, 0))
    return pl.pallas_call(
        pack_body,
        grid=(nblk,),
        in_specs=[lo_spec, hi_spec] * 4,
        out_specs=[out_spec] * 2,
        out_shape=[jax.ShapeDtypeStruct((_VH, 2 * _D), jnp.float32)] * 2,
    )(t0t, t0t, t1t, t1t, t2t, t2t, itt, itt)


def _sc_gather_score(users, pos, neg, utab, itab):
    mesh = plsc.VectorSubcoreMesh(core_axis_name="c", subcore_axis_name="s")

    rowbuf = pltpu.VMEM((_CHUNK, 2 * _D), jnp.float32)

    @functools.partial(
        pl.kernel,
        out_type=(
            jax.ShapeDtypeStruct((_L, _B), jnp.float32),
            jax.ShapeDtypeStruct((_NW, _L), jnp.float32),
        ),
        mesh=mesh,
        compiler_params=pltpu.CompilerParams(use_tc_tiling_on_sc=True,
                                             needs_layout_passes=False),
        scratch_types=[
            pltpu.VMEM((_BPW,), jnp.int32),
            pltpu.VMEM((_BPW,), jnp.int32),
            pltpu.VMEM((_BPW,), jnp.int32),
            pltpu.VMEM((_BPW,), jnp.int32),
            pltpu.VMEM((_BPW,), jnp.int32),
            pltpu.VMEM((_BPW,), jnp.int32),
            rowbuf, rowbuf, rowbuf,
            rowbuf, rowbuf, rowbuf,
            pltpu.VMEM((_L, _BPW), jnp.float32),
            pltpu.VMEM((_L,), jnp.float32),
            pltpu.SemaphoreType.DMA,
            pltpu.SemaphoreType.DMA,
        ],
    )
    def sc_body(users_h, pos_h, neg_h, utab_h, itab_h,
                q_h, r_h,
                uidx, pidx, nidx, muidx, mpidx, mnidx,
                uva, pva, nva, uvb, pvb, nvb,
                qv, regv, sema, semb):
        wid = lax.axis_index("s") * _NC + lax.axis_index("c")
        base = wid * _BPW
        pltpu.sync_copy(users_h.at[pl.ds(base, _BPW)], uidx)
        pltpu.sync_copy(pos_h.at[pl.ds(base, _BPW)], pidx)
        pltpu.sync_copy(neg_h.at[pl.ds(base, _BPW)], nidx)

        lane = lax.iota(jnp.int32, _L)

        def m_body(g, carry):
            sl16 = pl.ds(g * _L, _L)
            for src, dst in ((uidx, muidx), (pidx, mpidx), (nidx, mnidx)):
                v = src[sl16]
                dst[sl16] = v - jnp.where(v >= _VH, _VH, 0)
            return carry

        lax.fori_loop(0, _BPW // _L, m_body, 0)

        def fire(ci, uv, pv, nv, sem):
            isl = pl.ds(ci * _CHUNK, _CHUNK)
            pltpu.async_copy(utab_h.at[muidx.at[isl]], uv, sem)
            pltpu.async_copy(itab_h.at[mpidx.at[isl]], pv, sem)
            pltpu.async_copy(itab_h.at[mnidx.at[isl]], nv, sem)

        def drain(uv, pv, nv, sem):
            pltpu.make_async_copy(utab_h.at[pl.ds(0, _CHUNK)], uv, sem).wait()
            pltpu.make_async_copy(itab_h.at[pl.ds(0, _CHUNK)], pv, sem).wait()
            pltpu.make_async_copy(itab_h.at[pl.ds(0, _CHUNK)], nv, sem).wait()

        def compute(ci, uv, pv, nv, regacc):
            off = ci * _CHUNK

            def row_body(r, reg):
                rsplat = jnp.full((_L,), r, jnp.int32)
                isplat = rsplat + off
                cu = jnp.where(plsc.load_gather(uidx, [isplat]) >= _VH,
                               _D, 0) + lane
                cp = jnp.where(plsc.load_gather(pidx, [isplat]) >= _VH,
                               _D, 0) + lane
                cn = jnp.where(plsc.load_gather(nidx, [isplat]) >= _VH,
                               _D, 0) + lane
                q = None
                for k in range(_NBLK):
                    u = plsc.load_gather(uv, [rsplat, cu + k * _L])
                    pp = plsc.load_gather(pv, [rsplat, cp + k * _L])
                    nn = plsc.load_gather(nv, [rsplat, cn + k * _L])
                    term = u * (nn - pp)
                    q = term if q is None else q + term
                    reg = reg + (u * u + pp * pp + nn * nn)
                plsc.store_scatter(qv, [lane, isplat], q)
                return reg

            return plsc.parallel_loop(0, _CHUNK, unroll=4,
                                      carry=regacc)(row_body)

        regacc = jnp.zeros((_L,), jnp.float32)
        fire(0, uva, pva, nva, sema)
        fire(1, uvb, pvb, nvb, semb)
        drain(uva, pva, nva, sema)
        regacc = compute(0, uva, pva, nva, regacc)
        fire(2, uva, pva, nva, sema)
        drain(uvb, pvb, nvb, semb)
        regacc = compute(1, uvb, pvb, nvb, regacc)
        fire(3, uvb, pvb, nvb, semb)
        drain(uva, pva, nva, sema)
        regacc = compute(2, uva, pva, nva, regacc)
        drain(uvb, pvb, nvb, semb)
        regacc = compute(3, uvb, pvb, nvb, regacc)

        pltpu.sync_copy(qv, q_h.at[:, pl.ds(base, _BPW)])
        regv[...] = regacc
        pltpu.sync_copy(regv, r_h.at[wid])

    return sc_body(users, pos, neg, utab, itab)


def _tc_finalize(q, r):
    def tc_body(q_ref, r_ref, loss_ref, reg_ref):
        d = jnp.sum(q_ref[...], axis=0)
        loss_ref[0, 0] = jnp.mean(jax.nn.softplus(d))
        reg_ref[0, 0] = 0.5 * jnp.sum(r_ref[...]) / float(_B)

    loss, reg = pl.pallas_call(
        tc_body,
        out_shape=(
            jax.ShapeDtypeStruct((1, 1), jnp.float32),
            jax.ShapeDtypeStruct((1, 1), jnp.float32),
        ),
        out_specs=(
            pl.BlockSpec(memory_space=pltpu.SMEM),
            pl.BlockSpec(memory_space=pltpu.SMEM),
        ),
    )(q, r)
    return loss[0, 0], reg[0, 0]


def kernel(users, pos, neg, user_table_0, user_table_1, user_table_2,
           item_table):
    utab, itab = _tc_pack(user_table_0, user_table_1, user_table_2,
                          item_table)
    q, r = _sc_gather_score(users, pos, neg, utab, itab)
    loss, reg_loss = _tc_finalize(q, r)
    return (loss, reg_loss)
